# baseline (speedup 1.0000x reference)
import functools

import jax
import jax.numpy as jnp
from jax import lax
from jax.experimental import pallas as pl
from jax.experimental.pallas import tpu as pltpu
from jax.experimental.pallas import tpu_sc as plsc

B, L, W = 1024, 200, 512
NROWS = B * L
NW = 32
ROWS_PER_W = NROWS // NW
CHUNK = 128
NCHUNK = ROWS_PER_W // CHUNK
GROUPS = CHUNK // 16


def _rsqrt(v):
    i = lax.bitcast_convert_type(v, jnp.int32)
    y = lax.bitcast_convert_type(
        jnp.int32(0x5F3759DF) - lax.shift_right_arithmetic(i, 1), jnp.float32)
    for _ in range(3):
        y = y * (1.5 - 0.5 * v * y * y)
    return y


_mesh = plsc.VectorSubcoreMesh(core_axis_name="c", subcore_axis_name="s")


@functools.partial(
    pl.kernel,
    out_type=jax.ShapeDtypeStruct((NROWS, W), jnp.float32),
    mesh=_mesh,
    scratch_types=[
        pltpu.VMEM((60, W), jnp.float32),
        pltpu.VMEM((CHUNK, W), jnp.float32),
        pltpu.VMEM((CHUNK,), jnp.int32),
        pltpu.VMEM((W,), jnp.float32),
        pltpu.VMEM((W,), jnp.float32),
    ],
    compiler_params=pltpu.CompilerParams(needs_layout_passes=False),
)
def _sc_kernel(img_hbm, idx_hbm, tab_hbm, gamma_hbm, beta_hbm, out_hbm,
               tab_v, buf_v, idx_v, gamma_v, beta_v):
    wid = lax.axis_index("s") * 2 + lax.axis_index("c")
    base_w = wid * ROWS_PER_W
    pltpu.sync_copy(tab_hbm, tab_v)
    pltpu.sync_copy(gamma_hbm, gamma_v)
    pltpu.sync_copy(beta_hbm, beta_v)
    lanes = lax.iota(jnp.int32, 16)
    zeros_f = jnp.zeros((16,), jnp.float32)

    def chunk_body(ci, carry):
        base = base_w + ci * CHUNK
        pltpu.sync_copy(img_hbm.at[pl.ds(base, CHUNK)], buf_v)
        pltpu.sync_copy(idx_hbm.at[pl.ds(base, CHUNK)], idx_v)

        def group_body(g, carry2):
            r0 = g * 16
            rows = r0 + lanes
            deg = idx_v[pl.ds(r0, 16)]
            lpos = (base + rows) % L
            tidx = deg + (lpos >= (L // 2)).astype(jnp.int32) * 30

            def p1(j, c):
                s, q = c
                cj = jnp.broadcast_to(j, (16,)).astype(jnp.int32)
                x = plsc.load_gather(buf_v, [rows, cj])
                t = plsc.load_gather(tab_v, [tidx, cj])
                x = x + t
                plsc.store_scatter(buf_v, [rows, cj], x)
                return (s + x, q + x * x)

            s, q = lax.fori_loop(0, W, p1, (zeros_f, zeros_f))
            mean = s * (1.0 / W)
            var = q * (1.0 / W) - mean * mean
            rs = _rsqrt(var + 1e-5)

            def p2(j, c):
                cj = jnp.broadcast_to(j, (16,)).astype(jnp.int32)
                x = plsc.load_gather(buf_v, [rows, cj])
                gmm = plsc.load_gather(gamma_v, [cj])
                bta = plsc.load_gather(beta_v, [cj])
                y = (x - mean) * rs * gmm + bta
                plsc.store_scatter(buf_v, [rows, cj], y)
                return c

            lax.fori_loop(0, W, p2, 0)
            return carry2

        lax.fori_loop(0, GROUPS, group_body, 0)
        pltpu.sync_copy(buf_v, out_hbm.at[pl.ds(base, CHUNK)])
        return carry

    lax.fori_loop(0, NCHUNK, chunk_body, 0)


@jax.jit
def kernel(image_features, degrees, text_embed, degree_embedding,
           depth_embedding, ln_gamma, ln_beta):
    del text_embed
    img = image_features.reshape(NROWS, W)
    idx = degrees.reshape(NROWS)
    tab = jnp.concatenate([degree_embedding + depth_embedding[0][None, :],
                           degree_embedding + depth_embedding[1][None, :]], 0)
    out = _sc_kernel(img, idx, tab, ln_gamma, ln_beta)
    return out.reshape(B, L, W)


# speedup vs baseline: 1.5872x; 1.5872x over previous
import functools

import jax
import jax.numpy as jnp
from jax import lax
from jax.experimental import pallas as pl
from jax.experimental.pallas import tpu as pltpu
from jax.experimental.pallas import tpu_sc as plsc

B, L, W = 1024, 200, 512
NROWS = B * L
NW = 32
ROWS_PER_W = NROWS // NW
CHUNK = 128
NCHUNK = ROWS_PER_W // CHUNK
GROUPS = CHUNK // 16


def _rsqrt(v):
    i = lax.bitcast_convert_type(v, jnp.int32)
    y = lax.bitcast_convert_type(
        jnp.int32(0x5F3759DF) - lax.shift_right_arithmetic(i, 1), jnp.float32)
    for _ in range(3):
        y = y * (1.5 - 0.5 * v * y * y)
    return y


_mesh = plsc.VectorSubcoreMesh(core_axis_name="c", subcore_axis_name="s")


@functools.partial(
    pl.kernel,
    out_type=jax.ShapeDtypeStruct((NROWS, W), jnp.float32),
    mesh=_mesh,
    scratch_types=[
        pltpu.VMEM((60, W), jnp.float32),
        pltpu.VMEM((CHUNK, W), jnp.float32),
        pltpu.VMEM((CHUNK,), jnp.int32),
        pltpu.VMEM((W,), jnp.float32),
        pltpu.VMEM((W,), jnp.float32),
    ],
    compiler_params=pltpu.CompilerParams(needs_layout_passes=False),
)
def _sc_kernel(img_hbm, idx_hbm, tab_hbm, gamma_hbm, beta_hbm, out_hbm,
               tab_v, buf_v, idx_v, gamma_v, beta_v):
    wid = lax.axis_index("s") * 2 + lax.axis_index("c")
    base_w = wid * ROWS_PER_W
    pltpu.sync_copy(tab_hbm, tab_v)
    pltpu.sync_copy(gamma_hbm, gamma_v)
    pltpu.sync_copy(beta_hbm, beta_v)
    lanes = lax.iota(jnp.int32, 16)
    zeros_f = jnp.zeros((16,), jnp.float32)

    def chunk_body(ci, carry):
        base = base_w + ci * CHUNK
        pltpu.sync_copy(img_hbm.at[pl.ds(base, CHUNK)], buf_v)
        pltpu.sync_copy(idx_hbm.at[pl.ds(base, CHUNK)], idx_v)

        def group_body(g, carry2):
            r0 = g * 16
            rows = r0 + lanes
            deg = idx_v[pl.ds(r0, 16)]
            lpos = (base + rows) % L
            tidx = deg + (lpos >= (L // 2)).astype(jnp.int32) * 30

            @plsc.parallel_loop(0, W, unroll=8, carry=(zeros_f, zeros_f))
            def p1_out(j, c):
                s, q = c
                cj = jnp.broadcast_to(j, (16,)).astype(jnp.int32)
                x = plsc.load_gather(buf_v, [rows, cj])
                t = plsc.load_gather(tab_v, [tidx, cj])
                x = x + t
                plsc.store_scatter(buf_v, [rows, cj], x)
                return (s + x, q + x * x)

            s, q = p1_out
            mean = s * (1.0 / W)
            var = q * (1.0 / W) - mean * mean
            rs = _rsqrt(var + 1e-5)

            @plsc.parallel_loop(0, W, unroll=8)
            def p2(j):
                cj = jnp.broadcast_to(j, (16,)).astype(jnp.int32)
                x = plsc.load_gather(buf_v, [rows, cj])
                gmm = plsc.load_gather(gamma_v, [cj])
                bta = plsc.load_gather(beta_v, [cj])
                y = (x - mean) * rs * gmm + bta
                plsc.store_scatter(buf_v, [rows, cj], y)
            return carry2

        lax.fori_loop(0, GROUPS, group_body, 0)
        pltpu.sync_copy(buf_v, out_hbm.at[pl.ds(base, CHUNK)])
        return carry

    lax.fori_loop(0, NCHUNK, chunk_body, 0)


@jax.jit
def kernel(image_features, degrees, text_embed, degree_embedding,
           depth_embedding, ln_gamma, ln_beta):
    del text_embed
    img = image_features.reshape(NROWS, W)
    idx = degrees.reshape(NROWS)
    tab = jnp.concatenate([degree_embedding + depth_embedding[0][None, :],
                           degree_embedding + depth_embedding[1][None, :]], 0)
    out = _sc_kernel(img, idx, tab, ln_gamma, ln_beta)
    return out.reshape(B, L, W)


# speedup vs baseline: 8.9144x; 5.6164x over previous
import functools

import jax
import jax.numpy as jnp
from jax import lax
from jax.experimental import pallas as pl
from jax.experimental.pallas import tpu as pltpu
from jax.experimental.pallas import tpu_sc as plsc

B, L, W = 1024, 200, 512
NROWS = B * L
NW = 32
ROWS_PER_W = NROWS // NW
CHUNK = 128
NCHUNK = ROWS_PER_W // CHUNK
GROUPS = CHUNK // 16
JC = W // 16


def _rsqrt(v):
    i = lax.bitcast_convert_type(v, jnp.int32)
    y = lax.bitcast_convert_type(
        jnp.int32(0x5F3759DF) - lax.shift_right_arithmetic(i, 1), jnp.float32)
    for _ in range(3):
        y = y * (1.5 - 0.5 * v * y * y)
    return y


_mesh = plsc.VectorSubcoreMesh(core_axis_name="c", subcore_axis_name="s")


@functools.partial(
    pl.kernel,
    out_type=jax.ShapeDtypeStruct((NROWS, W), jnp.float32),
    mesh=_mesh,
    scratch_types=[
        pltpu.VMEM((CHUNK, W), jnp.float32),
        pltpu.VMEM((CHUNK,), jnp.int32),
        pltpu.VMEM((CHUNK,), jnp.int32),
        pltpu.VMEM((W,), jnp.float32),
        pltpu.VMEM((W,), jnp.float32),
        pltpu.SemaphoreType.DMA,
    ],
    compiler_params=pltpu.CompilerParams(needs_layout_passes=False),
)
def _sc_kernel(img_hbm, idx_hbm, tab_hbm, gamma_hbm, beta_hbm, out_hbm,
               buf_v, idx_v, tidx_v, gamma_v, beta_v, sem):
    wid = lax.axis_index("s") * 2 + lax.axis_index("c")
    base_w = wid * ROWS_PER_W
    pltpu.sync_copy(gamma_hbm, gamma_v)
    pltpu.sync_copy(beta_hbm, beta_v)
    lanes = lax.iota(jnp.int32, 16)
    zeros_f = jnp.zeros((16,), jnp.float32)

    def chunk_body(ci, carry):
        base = base_w + ci * CHUNK
        pltpu.sync_copy(img_hbm.at[pl.ds(base, CHUNK)], buf_v)
        pltpu.sync_copy(idx_hbm.at[pl.ds(base, CHUNK)], idx_v)
        for g in range(GROUPS):
            deg = idx_v[pl.ds(g * 16, 16)]
            lpos = (base + g * 16 + lanes) % L
            tidx_v[pl.ds(g * 16, 16)] = (
                deg + (lpos >= (L // 2)).astype(jnp.int32) * 30)
        pltpu.async_copy(tab_hbm.at[tidx_v], buf_v, sem, add=True).wait()

        @plsc.parallel_loop(0, CHUNK, unroll=2)
        def rows_loop(r):
            s = zeros_f
            q = zeros_f
            for jc in range(JC):
                x = buf_v[r, pl.ds(jc * 16, 16)]
                s = s + x
                q = q + x * x
            mean = jnp.broadcast_to(jnp.sum(s) * (1.0 / W), (16,))
            var = jnp.broadcast_to(jnp.sum(q) * (1.0 / W), (16,)) - mean * mean
            rs = _rsqrt(var + 1e-5)
            for jc in range(JC):
                x = buf_v[r, pl.ds(jc * 16, 16)]
                y = ((x - mean) * rs * gamma_v[pl.ds(jc * 16, 16)]
                     + beta_v[pl.ds(jc * 16, 16)])
                buf_v[r, pl.ds(jc * 16, 16)] = y

        pltpu.sync_copy(buf_v, out_hbm.at[pl.ds(base, CHUNK)])
        return carry

    lax.fori_loop(0, NCHUNK, chunk_body, 0)


@jax.jit
def kernel(image_features, degrees, text_embed, degree_embedding,
           depth_embedding, ln_gamma, ln_beta):
    del text_embed
    img = image_features.reshape(NROWS, W)
    idx = degrees.reshape(NROWS)
    tab = jnp.concatenate([degree_embedding + depth_embedding[0][None, :],
                           degree_embedding + depth_embedding[1][None, :]], 0)
    out = _sc_kernel(img, idx, tab, ln_gamma, ln_beta)
    return out.reshape(B, L, W)
